# baseline (speedup 1.0000x reference)
import jax
import jax.numpy as jnp
from jax.experimental import pallas as pl
from jax.experimental.pallas import tpu as pltpu

D_MODEL = 768
NUM_EXPERTS = 64
N_TOKENS = 32768
CH = 2048
NCH = N_TOKENS // CH


def _router_body(x_hbm, w_ref, b_ref, logits_hbm, probs_hbm,
                 xbuf, lbuf, pbuf, in_sem, lsem, psem):
    w = w_ref[...]
    b = b_ref[...]

    def in_copy(c, s):
        return pltpu.make_async_copy(
            x_hbm.at[pl.ds(c * CH, CH), :], xbuf.at[s], in_sem.at[s]
        )

    def l_copy(c, s):
        return pltpu.make_async_copy(
            lbuf.at[s], logits_hbm.at[pl.ds(c * CH, CH), :], lsem.at[s]
        )

    def p_copy(c, s):
        return pltpu.make_async_copy(
            pbuf.at[s], probs_hbm.at[pl.ds(c * CH, CH), :], psem.at[s]
        )

    in_copy(0, 0).start()
    in_copy(1, 1).start()
    for c in range(NCH):
        s = c % 2
        in_copy(c, s).wait()
        logits = jax.lax.dot_general(
            xbuf[s], w, (((1,), (1,)), ((), ())),
            preferred_element_type=jnp.float32,
        ) + b
        if c + 2 < NCH:
            in_copy(c + 2, s).start()
        if c >= 2:
            l_copy(c - 2, s).wait()
            p_copy(c - 2, s).wait()
        lbuf[s] = logits
        m = jnp.max(logits, axis=-1, keepdims=True)
        e = jnp.exp(logits - m)
        pbuf[s] = e / jnp.sum(e, axis=-1, keepdims=True)
        l_copy(c, s).start()
        p_copy(c, s).start()
    for c in (NCH - 2, NCH - 1):
        s = c % 2
        l_copy(c, s).wait()
        p_copy(c, s).wait()


def kernel(x, W, b):
    b2 = b.reshape(1, NUM_EXPERTS)
    out_shape = (
        jax.ShapeDtypeStruct((N_TOKENS, NUM_EXPERTS), jnp.float32),
        jax.ShapeDtypeStruct((N_TOKENS, NUM_EXPERTS), jnp.float32),
    )
    logits, probs = pl.pallas_call(
        _router_body,
        grid=(1,),
        in_specs=[
            pl.BlockSpec(memory_space=pltpu.MemorySpace.HBM),
            pl.BlockSpec((NUM_EXPERTS, D_MODEL), lambda i: (0, 0)),
            pl.BlockSpec((1, NUM_EXPERTS), lambda i: (0, 0)),
        ],
        out_specs=(
            pl.BlockSpec(memory_space=pltpu.MemorySpace.HBM),
            pl.BlockSpec(memory_space=pltpu.MemorySpace.HBM),
        ),
        out_shape=out_shape,
        scratch_shapes=[
            pltpu.VMEM((2, CH, D_MODEL), jnp.float32),
            pltpu.VMEM((2, CH, NUM_EXPERTS), jnp.float32),
            pltpu.VMEM((2, CH, NUM_EXPERTS), jnp.float32),
            pltpu.SemaphoreType.DMA((2,)),
            pltpu.SemaphoreType.DMA((2,)),
            pltpu.SemaphoreType.DMA((2,)),
        ],
        compiler_params=pltpu.CompilerParams(
            dimension_semantics=("arbitrary",),
        ),
    )(x, W, b2)
    return (logits, probs)


# speedup vs baseline: 1.0009x; 1.0009x over previous
import jax
import jax.numpy as jnp
from jax.experimental import pallas as pl
from jax.experimental.pallas import tpu as pltpu

D_MODEL = 768
NUM_EXPERTS = 64
N_TOKENS = 32768
CH = 2048
NCH = N_TOKENS // CH


def _router_body(x_hbm, w_ref, b_ref, logits_hbm, probs_hbm,
                 xbuf, lbuf, pbuf, in_sem, lsem, psem):
    w = w_ref[...]
    b = b_ref[...]

    def in_copy(c, s):
        return pltpu.make_async_copy(
            x_hbm.at[pl.ds(c * CH, CH), :], xbuf.at[s], in_sem.at[s]
        )

    def l_copy(c, s):
        return pltpu.make_async_copy(
            lbuf.at[s], logits_hbm.at[pl.ds(c * CH, CH), :], lsem.at[s]
        )

    def p_copy(c, s):
        return pltpu.make_async_copy(
            pbuf.at[s], probs_hbm.at[pl.ds(c * CH, CH), :], psem.at[s]
        )

    in_copy(0, 0).start()
    in_copy(1, 1).start()
    for c in range(NCH):
        s = c % 2
        in_copy(c, s).wait()
        logits = jax.lax.dot_general(
            xbuf[s], w, (((1,), (1,)), ((), ())),
            preferred_element_type=jnp.float32,
        ) + b
        if c + 2 < NCH:
            in_copy(c + 2, s).start()
        if c >= 2:
            l_copy(c - 2, s).wait()
            p_copy(c - 2, s).wait()
        lbuf[s] = logits
        m = jnp.max(logits, axis=-1, keepdims=True)
        e = jnp.exp(logits - m)
        pbuf[s] = e / jnp.sum(e, axis=-1, keepdims=True)
        l_copy(c, s).start(priority=1)
        p_copy(c, s).start(priority=1)
    for c in (NCH - 2, NCH - 1):
        s = c % 2
        l_copy(c, s).wait()
        p_copy(c, s).wait()


def kernel(x, W, b):
    b2 = b.reshape(1, NUM_EXPERTS)
    out_shape = (
        jax.ShapeDtypeStruct((N_TOKENS, NUM_EXPERTS), jnp.float32),
        jax.ShapeDtypeStruct((N_TOKENS, NUM_EXPERTS), jnp.float32),
    )
    logits, probs = pl.pallas_call(
        _router_body,
        grid=(1,),
        in_specs=[
            pl.BlockSpec(memory_space=pltpu.MemorySpace.HBM),
            pl.BlockSpec((NUM_EXPERTS, D_MODEL), lambda i: (0, 0)),
            pl.BlockSpec((1, NUM_EXPERTS), lambda i: (0, 0)),
        ],
        out_specs=(
            pl.BlockSpec(memory_space=pltpu.MemorySpace.HBM),
            pl.BlockSpec(memory_space=pltpu.MemorySpace.HBM),
        ),
        out_shape=out_shape,
        scratch_shapes=[
            pltpu.VMEM((2, CH, D_MODEL), jnp.float32),
            pltpu.VMEM((2, CH, NUM_EXPERTS), jnp.float32),
            pltpu.VMEM((2, CH, NUM_EXPERTS), jnp.float32),
            pltpu.SemaphoreType.DMA((2,)),
            pltpu.SemaphoreType.DMA((2,)),
            pltpu.SemaphoreType.DMA((2,)),
        ],
        compiler_params=pltpu.CompilerParams(
            dimension_semantics=("arbitrary",),
        ),
    )(x, W, b2)
    return (logits, probs)


# speedup vs baseline: 1.0781x; 1.0772x over previous
import jax
import jax.numpy as jnp
from jax.experimental import pallas as pl
from jax.experimental.pallas import tpu as pltpu

D_MODEL = 768
NUM_EXPERTS = 64
N_TOKENS = 32768
NSLICE = 4
BS = 1024
BT = NSLICE * BS


def _router_body(*refs):
    x_refs = refs[:NSLICE]
    w_ref, b_ref = refs[NSLICE], refs[NSLICE + 1]
    logits_ref, probs_ref = refs[NSLICE + 2], refs[NSLICE + 3]
    w = w_ref[...]
    b = b_ref[...]
    for k in range(NSLICE):
        logits = jax.lax.dot_general(
            x_refs[k][...], w, (((1,), (1,)), ((), ())),
            preferred_element_type=jnp.float32,
        )
        logits = logits + b
        logits_ref[pl.ds(k * BS, BS), :] = logits
        m = jnp.max(logits, axis=-1, keepdims=True)
        e = jnp.exp(logits - m)
        probs_ref[pl.ds(k * BS, BS), :] = e / jnp.sum(e, axis=-1, keepdims=True)


def kernel(x, W, b):
    b2 = b.reshape(1, NUM_EXPERTS)
    grid = (N_TOKENS // BT,)

    def x_map(k):
        return lambda i: (NSLICE * i + k, 0)

    in_specs = [pl.BlockSpec((BS, D_MODEL), x_map(k)) for k in range(NSLICE)]
    in_specs.append(pl.BlockSpec((NUM_EXPERTS, D_MODEL), lambda i: (0, 0)))
    in_specs.append(pl.BlockSpec((1, NUM_EXPERTS), lambda i: (0, 0)))

    out_specs = (
        pl.BlockSpec((BT, NUM_EXPERTS), lambda i: (i, 0)),
        pl.BlockSpec((BT, NUM_EXPERTS), lambda i: (i, 0)),
    )
    out_shape = (
        jax.ShapeDtypeStruct((N_TOKENS, NUM_EXPERTS), jnp.float32),
        jax.ShapeDtypeStruct((N_TOKENS, NUM_EXPERTS), jnp.float32),
    )
    logits, probs = pl.pallas_call(
        _router_body,
        grid=grid,
        in_specs=in_specs,
        out_specs=out_specs,
        out_shape=out_shape,
        compiler_params=pltpu.CompilerParams(
            dimension_semantics=("parallel",),
        ),
    )(*([x] * NSLICE), W, b2)
    return (logits, probs)
